# baseline (speedup 1.0000x reference)
import functools

import jax
import jax.numpy as jnp
from jax import lax
from jax.experimental import pallas as pl
from jax.experimental.pallas import tpu as pltpu
from jax.experimental.pallas import tpu_sc as plsc

NUM_FEATURES = 1000000
EMBED_DIM = 32
BATCH = 4096
FIELDS = 26
NW = 32
SPW = BATCH // NW
IPW = SPW * FIELDS
PPW = SPW * 32
NBLK = NUM_FEATURES // 128
BPW = NBLK // NW
BREM = NBLK - BPW * NW
TAIL_R = NBLK * 128
TAIL_W = (NUM_FEATURES - TAIL_R) * EMBED_DIM


def _p1_body(embT_h, tailf_h, idp_h, w_h, lin_h, wg_h,
             in0, in1, ob0, ob1, idxp_v, wbuf_v, sin, sout, semw):
    wid = lax.axis_index("s") * 2 + lax.axis_index("c")

    pltpu.sync_copy(idp_h.at[wid], idxp_v)

    def fire_w(k, _):
        o = pl.multiple_of(k * 16, 16)
        iv = idxp_v[pl.ds(o, 16)]
        pltpu.async_copy(w_h.at[iv], wbuf_v.at[pl.ds(o, 16)], semw)
        return 0

    lax.fori_loop(0, PPW // 16, fire_w, 0)

    start = wid * BPW + jnp.minimum(wid, BREM)
    cnt = jnp.where(wid < BREM, BPW + 1, BPW)
    lane32 = lax.iota(jnp.int32, 16) * 32

    def fire_in(b, buf):
        pltpu.async_copy(
            embT_h.at[:, pl.ds(pl.multiple_of(b * 128, 128), 128)], buf, sin)

    def wait_in(buf):
        pltpu.make_async_copy(embT_h.at[:, pl.ds(0, 128)], buf, sin).wait()

    def fire_out(b, obuf):
        pltpu.async_copy(
            obuf, lin_h.at[pl.ds(pl.multiple_of(b * 4096, 4096), 4096)], sout)

    def wait_out(obuf):
        pltpu.make_async_copy(lin_h.at[pl.ds(0, 4096)], obuf, sout).wait()

    fire_in(start, in0)

    def pair(g2, _):
        for par in (0, 1):
            bufi = in0 if par == 0 else in1
            bufo = ob0 if par == 0 else ob1
            b = start + g2 * 2 + par

            @pl.when(b < start + cnt)
            def _():
                wait_in(bufi)

                @pl.when(b + 1 < start + cnt)
                def _():
                    fire_in(b + 1, in1 if par == 0 else in0)

                for c in range(32):
                    for rg in range(8):
                        v = bufi[c, pl.ds(rg * 16, 16)]
                        plsc.store_scatter(
                            bufo, [lane32 + (rg * 512 + c)], v)

                @pl.when(b - 2 >= start)
                def _():
                    wait_out(bufo)

                fire_out(b, bufo)
        return 0

    lax.fori_loop(0, (BPW + 2) // 2, pair, 0)
    wait_out(ob0)
    wait_out(ob1)

    @pl.when(wid == 0)
    def _():
        pltpu.sync_copy(tailf_h, ob0.at[pl.ds(0, TAIL_W)])
        pltpu.sync_copy(ob0.at[pl.ds(0, TAIL_W)],
                        lin_h.at[pl.ds(TAIL_R * EMBED_DIM, TAIL_W)])

    pltpu.make_async_copy(w_h.at[pl.ds(0, PPW)], wbuf_v, semw).wait()
    pltpu.sync_copy(wbuf_v, wg_h.at[pl.ds(pl.multiple_of(wid * PPW, PPW), PPW)])


def _p2_body(emb_h, idf_h, wg_h, vp_h, bias_h, out_h,
             idx_v, rows_v, wg_v, vals_v, bias_v, out_v, sem_r):
    wid = lax.axis_index("s") * 2 + lax.axis_index("c")
    base = pl.multiple_of(wid * SPW, SPW)

    pltpu.sync_copy(idf_h.at[wid], idx_v)
    pltpu.sync_copy(wg_h.at[wid], wg_v)
    pltpu.sync_copy(vp_h.at[wid], vals_v)
    pltpu.sync_copy(bias_h, bias_v)

    def fire_rows(k, _):
        o = pl.multiple_of(k * 16, 16)
        iv = idx_v[pl.ds(o, 16)]
        pltpu.async_copy(emb_h.at[iv], rows_v.at[pl.ds(o, 16)], sem_r)
        return 0

    lax.fori_loop(0, IPW // 16, fire_rows, 0)
    pltpu.make_async_copy(emb_h.at[pl.ds(0, IPW)], rows_v, sem_r).wait()

    lane = lax.iota(jnp.int32, 16)
    perms = [lane ^ 1, lane ^ 2, lane ^ 4, lane ^ 8]

    def body(s, xvec):
        off = pl.multiple_of(s * 32, 32)
        v0 = vals_v[pl.ds(off, 16)]
        v1 = vals_v[pl.ds(off + 16, 16)]
        w0 = wg_v[pl.ds(off, 16)]
        w1 = wg_v[pl.ds(off + 16, 16)]
        i0 = s * FIELDS
        acc0 = jnp.zeros((16,), jnp.float32)
        acc1 = jnp.zeros((16,), jnp.float32)
        ssqv = jnp.zeros((16,), jnp.float32)
        for f in range(FIELDS):
            e0 = rows_v[i0 + f, pl.ds(0, 16)]
            e1 = rows_v[i0 + f, pl.ds(16, 16)]
            src = v0 if f < 16 else v1
            vb = src.at[jnp.full((16,), f % 16, jnp.int32)].get(
                mode="promise_in_bounds")
            s0 = e0 * vb
            s1 = e1 * vb
            acc0 = acc0 + s0
            acc1 = acc1 + s1
            ssqv = ssqv + (s0 * s0 + s1 * s1)
        sqv = acc0 * acc0 + acc1 * acc1
        linv = w0 * v0 + w1 * v1
        xv = linv + 0.5 * (sqv - ssqv)
        for p in perms:
            xv = xv + xv.at[p].get(mode="promise_in_bounds")
        xvec = jnp.where(lane == (s % 16), xv, xvec)

        @pl.when(s % 16 == 15)
        def _():
            out_v[pl.ds(pl.multiple_of((s // 16) * 16, 16), 16)] = xvec

        return xvec

    lax.fori_loop(0, SPW, body, jnp.zeros((16,), jnp.float32))

    bvec = bias_v[...]
    for k in range(SPW // 16):
        x = out_v[pl.ds(k * 16, 16)]
        y = 1.0 / (1.0 + jnp.exp(-(x + bvec)))
        out_v[pl.ds(k * 16, 16)] = y
    pltpu.sync_copy(out_v, out_h.at[pl.ds(base, SPW)])


@jax.jit
def _fm(emb, linear_w, idf, idp, vp, bias16, tailf):
    mesh = plsc.VectorSubcoreMesh(core_axis_name="c", subcore_axis_name="s")
    p1 = functools.partial(
        pl.kernel,
        mesh=mesh,
        out_type=(
            jax.ShapeDtypeStruct((NUM_FEATURES * EMBED_DIM,), jnp.float32),
            jax.ShapeDtypeStruct((NW * PPW,), jnp.float32),
        ),
        scratch_types=[
            pltpu.VMEM((32, 128), jnp.float32),
            pltpu.VMEM((32, 128), jnp.float32),
            pltpu.VMEM((4096,), jnp.float32),
            pltpu.VMEM((4096,), jnp.float32),
            pltpu.VMEM((PPW,), jnp.int32),
            pltpu.VMEM((PPW,), jnp.float32),
            pltpu.SemaphoreType.DMA,
            pltpu.SemaphoreType.DMA,
            pltpu.SemaphoreType.DMA,
        ],
        compiler_params=pltpu.CompilerParams(
            use_tc_tiling_on_sc=True, needs_layout_passes=False),
    )(_p1_body)
    lin, wg_all = p1(emb.T, tailf, idp, linear_w)

    p2 = functools.partial(
        pl.kernel,
        mesh=mesh,
        out_type=jax.ShapeDtypeStruct((BATCH,), jnp.float32),
        scratch_types=[
            pltpu.VMEM((IPW,), jnp.int32),
            pltpu.VMEM((IPW, EMBED_DIM), jnp.float32),
            pltpu.VMEM((PPW,), jnp.float32),
            pltpu.VMEM((PPW,), jnp.float32),
            pltpu.VMEM((16,), jnp.float32),
            pltpu.VMEM((SPW,), jnp.float32),
            pltpu.SemaphoreType.DMA,
        ],
        compiler_params=pltpu.CompilerParams(use_tc_tiling_on_sc=False),
    )(_p2_body)
    return p2(lin.reshape(NUM_FEATURES, EMBED_DIM),
              idf, wg_all.reshape(NW, PPW), vp, bias16)


def kernel(feature_ids_batch, feature_values_batch, bias, linear_w, emb):
    ids = feature_ids_batch.astype(jnp.int32)
    vals = feature_values_batch.astype(jnp.float32)
    pad_i = jnp.zeros((BATCH, 32 - FIELDS), jnp.int32)
    pad_v = jnp.zeros((BATCH, 32 - FIELDS), jnp.float32)
    idf = ids.reshape(NW, IPW)
    idp = jnp.concatenate([ids, pad_i], axis=1).reshape(NW, PPW)
    vp = jnp.concatenate([vals, pad_v], axis=1).reshape(NW, PPW)
    bias16 = jnp.broadcast_to(bias, (16,))
    tailf = emb[TAIL_R:, :].reshape(-1)
    out = _fm(emb, linear_w, idf, idp, vp, bias16, tailf)
    return out.reshape(BATCH, 1)


# speedup vs baseline: 1.1984x; 1.1984x over previous
import functools

import jax
import jax.numpy as jnp
from jax import lax
from jax.experimental import pallas as pl
from jax.experimental.pallas import tpu as pltpu
from jax.experimental.pallas import tpu_sc as plsc

NUM_FEATURES = 1000000
EMBED_DIM = 32
BATCH = 4096
FIELDS = 26
NW = 32
SPW = BATCH // NW
IPW = SPW * FIELDS
PPW = SPW * 32
TCOLS = 2048
TGRID = (NUM_FEATURES + TCOLS - 1) // TCOLS
WCH = 16
WCS = PPW // WCH


def _tr_body(x_ref, o_ref):
    y = x_ref[...].T.reshape(TCOLS // 4, 4, EMBED_DIM)
    o_ref[...] = jnp.concatenate(
        [y[:, b, :] for b in range(4)], axis=1)


@jax.jit
def _fm(emb, linear_w, idf, qidp, wlo, vp, bias16):
    lin = pl.pallas_call(
        _tr_body,
        grid=(TGRID,),
        in_specs=[pl.BlockSpec((EMBED_DIM, TCOLS), lambda i: (0, i))],
        out_specs=pl.BlockSpec((TCOLS * EMBED_DIM // 128, 128),
                               lambda i: (i, 0)),
        out_shape=jax.ShapeDtypeStruct(
            (NUM_FEATURES * EMBED_DIM // 128, 128), jnp.float32),
    )(emb.T)
    lin32 = lin.reshape(NUM_FEATURES, EMBED_DIM)

    p2 = functools.partial(
        pl.kernel,
        mesh=plsc.VectorSubcoreMesh(core_axis_name="c", subcore_axis_name="s"),
        out_type=jax.ShapeDtypeStruct((BATCH,), jnp.float32),
        scratch_types=[
            pltpu.VMEM((IPW,), jnp.int32),
            pltpu.VMEM((IPW, EMBED_DIM), jnp.float32),
            pltpu.VMEM((PPW,), jnp.int32),
            pltpu.VMEM((PPW,), jnp.int32),
            pltpu.VMEM((WCS, 16), jnp.float32),
            pltpu.VMEM((PPW,), jnp.float32),
            pltpu.VMEM((PPW,), jnp.float32),
            pltpu.VMEM((16,), jnp.float32),
            pltpu.VMEM((SPW,), jnp.float32),
            pltpu.SemaphoreType.DMA,
            pltpu.SemaphoreType.DMA,
        ],
        compiler_params=pltpu.CompilerParams(
            use_tc_tiling_on_sc=False, needs_layout_passes=False),
    )(_p2_body)
    return p2(lin32, linear_w.reshape(NUM_FEATURES // 16, 16),
              idf, qidp, wlo, vp, bias16)


def _p2_body(emb_h, w16_h, idf_h, qidp_h, wlo_h, vp_h, bias_h, out_h,
             idx_v, rows_v, qidp_v, wlo_v, w16_v, wg_v, vals_v, bias_v,
             out_v, sem_r, sem_w):
    wid = lax.axis_index("s") * 2 + lax.axis_index("c")
    base = pl.multiple_of(wid * SPW, SPW)

    pltpu.sync_copy(idf_h.at[wid], idx_v)
    pltpu.sync_copy(qidp_h.at[wid], qidp_v)
    pltpu.sync_copy(wlo_h.at[wid], wlo_v)
    pltpu.sync_copy(vp_h.at[wid], vals_v)
    pltpu.sync_copy(bias_h, bias_v)

    lane = lax.iota(jnp.int32, 16)

    def fire_rows(k, _):
        o = pl.multiple_of(k * 16, 16)
        iv = idx_v[pl.ds(o, 16)]
        pltpu.async_copy(emb_h.at[iv], rows_v.at[pl.ds(o, 16)], sem_r)
        return 0

    lax.fori_loop(0, IPW // 16, fire_rows, 0)

    def wchunk(c, _):
        cb = pl.multiple_of(c * WCS, WCS)
        for g in range(WCS // 16):
            iv = qidp_v[pl.ds(cb + g * 16, 16)]
            pltpu.async_copy(w16_h.at[iv], w16_v.at[pl.ds(g * 16, 16)], sem_w)
        pltpu.make_async_copy(w16_h.at[pl.ds(0, WCS)], w16_v, sem_w).wait()
        for g in range(WCS // 16):
            colv = wlo_v[pl.ds(cb + g * 16, 16)]
            wv = plsc.load_gather(w16_v, [g * 16 + lane, colv])
            wg_v[pl.ds(cb + g * 16, 16)] = wv
        return 0

    lax.fori_loop(0, WCH, wchunk, 0)
    pltpu.make_async_copy(emb_h.at[pl.ds(0, IPW)], rows_v, sem_r).wait()

    perms = [lane ^ 1, lane ^ 2, lane ^ 4, lane ^ 8]

    def body(s, xvec):
        off = pl.multiple_of(s * 32, 32)
        v0 = vals_v[pl.ds(off, 16)]
        v1 = vals_v[pl.ds(off + 16, 16)]
        w0 = wg_v[pl.ds(off, 16)]
        w1 = wg_v[pl.ds(off + 16, 16)]
        i0 = s * FIELDS
        acc0 = jnp.zeros((16,), jnp.float32)
        acc1 = jnp.zeros((16,), jnp.float32)
        ssqv = jnp.zeros((16,), jnp.float32)
        for f in range(FIELDS):
            e0 = rows_v[i0 + f, pl.ds(0, 16)]
            e1 = rows_v[i0 + f, pl.ds(16, 16)]
            src = v0 if f < 16 else v1
            vb = src.at[jnp.full((16,), f % 16, jnp.int32)].get(
                mode="promise_in_bounds")
            s0 = e0 * vb
            s1 = e1 * vb
            acc0 = acc0 + s0
            acc1 = acc1 + s1
            ssqv = ssqv + (s0 * s0 + s1 * s1)
        sqv = acc0 * acc0 + acc1 * acc1
        linv = w0 * v0 + w1 * v1
        xv = linv + 0.5 * (sqv - ssqv)
        for p in perms:
            xv = xv + xv.at[p].get(mode="promise_in_bounds")
        xvec = jnp.where(lane == (s % 16), xv, xvec)

        @pl.when(s % 16 == 15)
        def _():
            out_v[pl.ds(pl.multiple_of((s // 16) * 16, 16), 16)] = xvec

        return xvec

    lax.fori_loop(0, SPW, body, jnp.zeros((16,), jnp.float32))

    bvec = bias_v[...]
    for k in range(SPW // 16):
        x = out_v[pl.ds(k * 16, 16)]
        y = 1.0 / (1.0 + jnp.exp(-(x + bvec)))
        out_v[pl.ds(k * 16, 16)] = y
    pltpu.sync_copy(out_v, out_h.at[pl.ds(base, SPW)])


def kernel(feature_ids_batch, feature_values_batch, bias, linear_w, emb):
    ids = feature_ids_batch.astype(jnp.int32)
    vals = feature_values_batch.astype(jnp.float32)
    pad_i = jnp.zeros((BATCH, 32 - FIELDS), jnp.int32)
    pad_v = jnp.zeros((BATCH, 32 - FIELDS), jnp.float32)
    idf = ids.reshape(NW, IPW)
    idp = jnp.concatenate([ids, pad_i], axis=1).reshape(NW, PPW)
    qidp = idp >> 4
    wlo = idp & 15
    vp = jnp.concatenate([vals, pad_v], axis=1).reshape(NW, PPW)
    bias16 = jnp.broadcast_to(bias, (16,))
    out = _fm(emb, linear_w, idf, qidp, wlo, vp, bias16)
    return out.reshape(BATCH, 1)


# speedup vs baseline: 2.4010x; 2.0035x over previous
import functools

import jax
import jax.numpy as jnp
from jax import lax
from jax.experimental import pallas as pl
from jax.experimental.pallas import tpu as pltpu
from jax.experimental.pallas import tpu_sc as plsc

NUM_FEATURES = 1000000
EMBED_DIM = 32
BATCH = 4096
FIELDS = 26
NW = 32
SPW = BATCH // NW
IPW = SPW * FIELDS
PPW = SPW * 32
NBLK = NUM_FEATURES // 128
BPW = NBLK // NW
BREM = NBLK - BPW * NW
TAIL_R = NBLK * 128
TAIL_W = (NUM_FEATURES - TAIL_R) * EMBED_DIM
WSTRIPE = 62496
WLAST = NUM_FEATURES - 15 * WSTRIPE


def _p1_body(embT_h, tailf_h, w_h, idp_h, lin_h, wg_h,
             in0, in1, ob0, ob1, idp_v, wbuf_v, wbn_v, wsp, sin, sout, semw):
    wid = lax.axis_index("s") * 2 + lax.axis_index("c")
    sid = lax.axis_index("s")
    start = wid * BPW + jnp.minimum(wid, BREM)
    cnt = jnp.where(wid < BREM, BPW + 1, BPW)
    lane = lax.iota(jnp.int32, 16)

    @pl.when(sid < 15)
    def _():
        for j in range(4):
            o = pl.multiple_of(sid * 65536 + j * 16384, 16384)
            pltpu.sync_copy(w_h.at[pl.ds(o, 16384)], wbn_v)
            pltpu.sync_copy(wbn_v, wsp.at[pl.ds(o, 16384)])

    @pl.when(sid == 15)
    def _():
        pltpu.sync_copy(w_h.at[pl.ds(983040, 16384)], wbn_v)
        pltpu.sync_copy(wbn_v, wsp.at[pl.ds(983040, 16384)])
        pltpu.sync_copy(w_h.at[pl.ds(999424, 576)], wbn_v.at[pl.ds(0, 576)])
        pltpu.sync_copy(wbn_v.at[pl.ds(0, 576)], wsp.at[pl.ds(999424, 576)])

    pltpu.sync_copy(idp_h.at[wid], idp_v)
    plsc.subcore_barrier()

    def fire_w(k, _):
        o = pl.multiple_of(k * 16, 16)
        iv = idp_v[pl.ds(o, 16)]
        pltpu.async_copy(wsp.at[iv], wbuf_v.at[pl.ds(o, 16)], semw)
        return 0

    lax.fori_loop(0, PPW // 16, fire_w, 0)

    def fire_in(b, buf):
        pltpu.async_copy(
            embT_h.at[:, pl.ds(pl.multiple_of(b * 128, 128), 128)], buf, sin)

    def wait_in(buf):
        pltpu.make_async_copy(embT_h.at[:, pl.ds(0, 128)], buf, sin).wait()

    def fire_out(b, obuf):
        pltpu.async_copy(
            obuf, lin_h.at[pl.ds(pl.multiple_of(b * 4096, 4096), 4096)], sout)

    def wait_out(obuf):
        pltpu.make_async_copy(lin_h.at[pl.ds(0, 4096)], obuf, sout).wait()

    fire_in(start, in0)

    def pair(g2, _):
        for par in (0, 1):
            bufi = in0 if par == 0 else in1
            bufo = ob0 if par == 0 else ob1
            b = start + g2 * 2 + par

            @pl.when(b < start + cnt)
            def _():
                wait_in(bufi)

                @pl.when(b + 1 < start + cnt)
                def _():
                    fire_in(b + 1, in1 if par == 0 else in0)

                def c0loop(c0, _):
                    cv = (c0 + lane) & 31
                    for rg in range(8):
                        rv = rg * 16 + lane
                        e = plsc.load_gather(bufi, [cv, rv])
                        plsc.store_scatter(bufo, [rv * 32 + cv], e)
                    return 0

                lax.fori_loop(0, 32, c0loop, 0)

                @pl.when(b - 2 >= start)
                def _():
                    wait_out(bufo)

                fire_out(b, bufo)
        return 0

    lax.fori_loop(0, (BPW + 2) // 2, pair, 0)
    wait_out(ob0)
    wait_out(ob1)

    @pl.when(wid == 0)
    def _():
        pltpu.sync_copy(tailf_h, ob0.at[pl.ds(0, TAIL_W)])
        pltpu.sync_copy(ob0.at[pl.ds(0, TAIL_W)],
                        lin_h.at[pl.ds(TAIL_R * EMBED_DIM, TAIL_W)])

    pltpu.make_async_copy(wsp.at[pl.ds(0, PPW)], wbuf_v, semw).wait()
    pltpu.sync_copy(wbuf_v, wg_h.at[pl.ds(pl.multiple_of(wid * PPW, PPW), PPW)])


def _p2_body(emb_h, wg2_h, idf_h, vp_h, bias_h, out_h,
             idx_v, rows_v, wg_v, vals_v, bias_v, out_v, sem_r):
    wid = lax.axis_index("s") * 2 + lax.axis_index("c")
    base = pl.multiple_of(wid * SPW, SPW)

    pltpu.sync_copy(idf_h.at[wid], idx_v)
    pltpu.sync_copy(wg2_h.at[wid], wg_v)
    pltpu.sync_copy(vp_h.at[wid], vals_v)
    pltpu.sync_copy(bias_h, bias_v)

    def fire_rows(k, _):
        o = pl.multiple_of(k * 16, 16)
        iv = idx_v[pl.ds(o, 16)]
        pltpu.async_copy(emb_h.at[iv], rows_v.at[pl.ds(o, 16)], sem_r)
        return 0

    lax.fori_loop(0, IPW // 16, fire_rows, 0)
    pltpu.make_async_copy(emb_h.at[pl.ds(0, IPW)], rows_v, sem_r).wait()

    lane = lax.iota(jnp.int32, 16)
    perms = [lane ^ 1, lane ^ 2, lane ^ 4, lane ^ 8]

    def body(s, xvec):
        off = pl.multiple_of(s * 32, 32)
        v0 = vals_v[pl.ds(off, 16)]
        v1 = vals_v[pl.ds(off + 16, 16)]
        w0 = wg_v[pl.ds(off, 16)]
        w1 = wg_v[pl.ds(off + 16, 16)]
        i0 = s * FIELDS
        acc0 = jnp.zeros((16,), jnp.float32)
        acc1 = jnp.zeros((16,), jnp.float32)
        ssqv = jnp.zeros((16,), jnp.float32)
        for f in range(FIELDS):
            e0 = rows_v[i0 + f, pl.ds(0, 16)]
            e1 = rows_v[i0 + f, pl.ds(16, 16)]
            src = v0 if f < 16 else v1
            vb = src.at[jnp.full((16,), f % 16, jnp.int32)].get(
                mode="promise_in_bounds")
            s0 = e0 * vb
            s1 = e1 * vb
            acc0 = acc0 + s0
            acc1 = acc1 + s1
            ssqv = ssqv + (s0 * s0 + s1 * s1)
        sqv = acc0 * acc0 + acc1 * acc1
        linv = w0 * v0 + w1 * v1
        xv = linv + 0.5 * (sqv - ssqv)
        for p in perms:
            xv = xv + xv.at[p].get(mode="promise_in_bounds")
        xvec = jnp.where(lane == (s % 16), xv, xvec)

        @pl.when(s % 16 == 15)
        def _():
            out_v[pl.ds(pl.multiple_of((s // 16) * 16, 16), 16)] = xvec

        return xvec

    lax.fori_loop(0, SPW, body, jnp.zeros((16,), jnp.float32))

    bvec = bias_v[...]
    for k in range(SPW // 16):
        x = out_v[pl.ds(k * 16, 16)]
        y = 1.0 / (1.0 + jnp.exp(-(x + bvec)))
        out_v[pl.ds(k * 16, 16)] = y
    pltpu.sync_copy(out_v, out_h.at[pl.ds(base, SPW)])


@jax.jit
def _fm(emb, linear_w, idf, idp, vp, bias16, tailf):
    mesh = plsc.VectorSubcoreMesh(core_axis_name="c", subcore_axis_name="s")
    p1 = functools.partial(
        pl.kernel,
        mesh=mesh,
        out_type=(
            jax.ShapeDtypeStruct((NUM_FEATURES * EMBED_DIM,), jnp.float32),
            jax.ShapeDtypeStruct((NW * PPW,), jnp.float32),
        ),
        scratch_types=[
            pltpu.VMEM((32, 128), jnp.float32),
            pltpu.VMEM((32, 128), jnp.float32),
            pltpu.VMEM((4096,), jnp.float32),
            pltpu.VMEM((4096,), jnp.float32),
            pltpu.VMEM((PPW,), jnp.int32),
            pltpu.VMEM((PPW,), jnp.float32),
            pltpu.VMEM((16384,), jnp.float32),
            pltpu.VMEM_SHARED((NUM_FEATURES,), jnp.float32),
            pltpu.SemaphoreType.DMA,
            pltpu.SemaphoreType.DMA,
            pltpu.SemaphoreType.DMA,
        ],
        compiler_params=pltpu.CompilerParams(
            use_tc_tiling_on_sc=True, needs_layout_passes=False),
    )(_p1_body)
    lin, wg_all = p1(emb.T, tailf, linear_w, idp)

    p2 = functools.partial(
        pl.kernel,
        mesh=mesh,
        out_type=jax.ShapeDtypeStruct((BATCH,), jnp.float32),
        scratch_types=[
            pltpu.VMEM((IPW,), jnp.int32),
            pltpu.VMEM((IPW, EMBED_DIM), jnp.float32),
            pltpu.VMEM((PPW,), jnp.float32),
            pltpu.VMEM((PPW,), jnp.float32),
            pltpu.VMEM((16,), jnp.float32),
            pltpu.VMEM((SPW,), jnp.float32),
            pltpu.SemaphoreType.DMA,
        ],
        compiler_params=pltpu.CompilerParams(
            use_tc_tiling_on_sc=False, needs_layout_passes=False),
    )(_p2_body)
    return p2(lin.reshape(NUM_FEATURES, EMBED_DIM),
              wg_all.reshape(NW, PPW), idf, vp, bias16)


def kernel(feature_ids_batch, feature_values_batch, bias, linear_w, emb):
    ids = feature_ids_batch.astype(jnp.int32)
    vals = feature_values_batch.astype(jnp.float32)
    pad_i = jnp.zeros((BATCH, 32 - FIELDS), jnp.int32)
    pad_v = jnp.zeros((BATCH, 32 - FIELDS), jnp.float32)
    idf = ids.reshape(NW, IPW)
    idp = jnp.concatenate([ids, pad_i], axis=1).reshape(NW, PPW)
    vp = jnp.concatenate([vals, pad_v], axis=1).reshape(NW, PPW)
    bias16 = jnp.broadcast_to(bias, (16,))
    tailf = emb[TAIL_R:, :].reshape(-1)
    out = _fm(emb, linear_w, idf, idp, vp, bias16, tailf)
    return out.reshape(BATCH, 1)


# speedup vs baseline: 2.4882x; 1.0363x over previous
import functools

import jax
import jax.numpy as jnp
from jax import lax
from jax.experimental import pallas as pl
from jax.experimental.pallas import tpu as pltpu
from jax.experimental.pallas import tpu_sc as plsc

NUM_FEATURES = 1000000
EMBED_DIM = 32
BATCH = 4096
FIELDS = 26
NW = 32
SPW = BATCH // NW
IPW = SPW * FIELDS
PPW = SPW * 32
NBLK = NUM_FEATURES // 128
BPW = NBLK // NW
BREM = NBLK - BPW * NW
TAIL_R = NBLK * 128
TAIL_W = (NUM_FEATURES - TAIL_R) * EMBED_DIM
WSTRIPE = 62496
WLAST = NUM_FEATURES - 15 * WSTRIPE


def _p1_body(embT_h, tailf_h, w_h, idp_h, lin_h, wg_h,
             in0, in1, ob0, ob1, idp_v, wbuf_v, wbn_v, wsp, sin, sout, semw):
    wid = lax.axis_index("s") * 2 + lax.axis_index("c")
    sid = lax.axis_index("s")
    start = wid * BPW + jnp.minimum(wid, BREM)
    cnt = jnp.where(wid < BREM, BPW + 1, BPW)
    lane = lax.iota(jnp.int32, 16)

    @pl.when(sid < 15)
    def _():
        for j in range(4):
            o = pl.multiple_of(sid * 65536 + j * 16384, 16384)
            pltpu.sync_copy(w_h.at[pl.ds(o, 16384)], wbn_v)
            pltpu.sync_copy(wbn_v, wsp.at[pl.ds(o, 16384)])

    @pl.when(sid == 15)
    def _():
        pltpu.sync_copy(w_h.at[pl.ds(983040, 16384)], wbn_v)
        pltpu.sync_copy(wbn_v, wsp.at[pl.ds(983040, 16384)])
        pltpu.sync_copy(w_h.at[pl.ds(999424, 576)], wbn_v.at[pl.ds(0, 576)])
        pltpu.sync_copy(wbn_v.at[pl.ds(0, 576)], wsp.at[pl.ds(999424, 576)])

    pltpu.sync_copy(idp_h.at[wid], idp_v)
    plsc.subcore_barrier()

    def fire_w(k, _):
        o = pl.multiple_of(k * 16, 16)
        iv = idp_v[pl.ds(o, 16)]
        pltpu.async_copy(wsp.at[iv], wbuf_v.at[pl.ds(o, 16)], semw)
        return 0

    lax.fori_loop(0, PPW // 16, fire_w, 0)

    def fire_in(b, buf):
        pltpu.async_copy(
            embT_h.at[:, pl.ds(pl.multiple_of(b * 128, 128), 128)], buf, sin)

    def wait_in(buf):
        pltpu.make_async_copy(embT_h.at[:, pl.ds(0, 128)], buf, sin).wait()

    def fire_out(b, obuf):
        pltpu.async_copy(
            obuf, lin_h.at[pl.ds(pl.multiple_of(b * 4096, 4096), 4096)], sout)

    def wait_out(obuf):
        pltpu.make_async_copy(lin_h.at[pl.ds(0, 4096)], obuf, sout).wait()

    fire_in(start, in0)

    def pair(g2, _):
        for par in (0, 1):
            bufi = in0 if par == 0 else in1
            bufo = ob0 if par == 0 else ob1
            b = start + g2 * 2 + par

            @pl.when(b < start + cnt)
            def _():
                wait_in(bufi)

                @pl.when(b + 1 < start + cnt)
                def _():
                    fire_in(b + 1, in1 if par == 0 else in0)

                def c0loop(c04, _):
                    for u in range(4):
                        cv = (c04 * 4 + u + lane) & 31
                        for rg in range(8):
                            rv = rg * 16 + lane
                            e = plsc.load_gather(bufi, [cv, rv])
                            plsc.store_scatter(bufo, [rv * 32 + cv], e)
                    return 0

                lax.fori_loop(0, 8, c0loop, 0)

                @pl.when(b - 2 >= start)
                def _():
                    wait_out(bufo)

                fire_out(b, bufo)
        return 0

    lax.fori_loop(0, (BPW + 2) // 2, pair, 0)
    wait_out(ob0)
    wait_out(ob1)

    @pl.when(wid == 0)
    def _():
        pltpu.sync_copy(tailf_h, ob0.at[pl.ds(0, TAIL_W)])
        pltpu.sync_copy(ob0.at[pl.ds(0, TAIL_W)],
                        lin_h.at[pl.ds(TAIL_R * EMBED_DIM, TAIL_W)])

    pltpu.make_async_copy(wsp.at[pl.ds(0, PPW)], wbuf_v, semw).wait()
    pltpu.sync_copy(wbuf_v, wg_h.at[pl.ds(pl.multiple_of(wid * PPW, PPW), PPW)])


def _p2_body(emb_h, wg2_h, idf_h, vp_h, bias_h, out_h,
             idx_v, rows_v, wg_v, vals_v, bias_v, out_v, sem_r):
    wid = lax.axis_index("s") * 2 + lax.axis_index("c")
    base = pl.multiple_of(wid * SPW, SPW)

    pltpu.sync_copy(idf_h.at[wid], idx_v)
    pltpu.sync_copy(wg2_h.at[wid], wg_v)
    pltpu.sync_copy(vp_h.at[wid], vals_v)
    pltpu.sync_copy(bias_h, bias_v)

    def fire_rows(k, _):
        o = pl.multiple_of(k * 16, 16)
        iv = idx_v[pl.ds(o, 16)]
        pltpu.async_copy(emb_h.at[iv], rows_v.at[pl.ds(o, 16)], sem_r)
        return 0

    lax.fori_loop(0, IPW // 16, fire_rows, 0)
    pltpu.make_async_copy(emb_h.at[pl.ds(0, IPW)], rows_v, sem_r).wait()

    lane = lax.iota(jnp.int32, 16)
    perms = [lane ^ 1, lane ^ 2, lane ^ 4, lane ^ 8]

    def body(s, xvec):
        off = pl.multiple_of(s * 32, 32)
        v0 = vals_v[pl.ds(off, 16)]
        v1 = vals_v[pl.ds(off + 16, 16)]
        w0 = wg_v[pl.ds(off, 16)]
        w1 = wg_v[pl.ds(off + 16, 16)]
        i0 = s * FIELDS
        acc0 = jnp.zeros((16,), jnp.float32)
        acc1 = jnp.zeros((16,), jnp.float32)
        ssqv = jnp.zeros((16,), jnp.float32)
        for f in range(FIELDS):
            e0 = rows_v[i0 + f, pl.ds(0, 16)]
            e1 = rows_v[i0 + f, pl.ds(16, 16)]
            src = v0 if f < 16 else v1
            vb = src.at[jnp.full((16,), f % 16, jnp.int32)].get(
                mode="promise_in_bounds")
            s0 = e0 * vb
            s1 = e1 * vb
            acc0 = acc0 + s0
            acc1 = acc1 + s1
            ssqv = ssqv + (s0 * s0 + s1 * s1)
        sqv = acc0 * acc0 + acc1 * acc1
        linv = w0 * v0 + w1 * v1
        xv = linv + 0.5 * (sqv - ssqv)
        for p in perms:
            xv = xv + xv.at[p].get(mode="promise_in_bounds")
        xvec = jnp.where(lane == (s % 16), xv, xvec)

        @pl.when(s % 16 == 15)
        def _():
            out_v[pl.ds(pl.multiple_of((s // 16) * 16, 16), 16)] = xvec

        return xvec

    lax.fori_loop(0, SPW, body, jnp.zeros((16,), jnp.float32))

    bvec = bias_v[...]
    for k in range(SPW // 16):
        x = out_v[pl.ds(k * 16, 16)]
        y = 1.0 / (1.0 + jnp.exp(-(x + bvec)))
        out_v[pl.ds(k * 16, 16)] = y
    pltpu.sync_copy(out_v, out_h.at[pl.ds(base, SPW)])


@jax.jit
def _fm(emb, linear_w, idf, idp, vp, bias16, tailf):
    mesh = plsc.VectorSubcoreMesh(core_axis_name="c", subcore_axis_name="s")
    p1 = functools.partial(
        pl.kernel,
        mesh=mesh,
        out_type=(
            jax.ShapeDtypeStruct((NUM_FEATURES * EMBED_DIM,), jnp.float32),
            jax.ShapeDtypeStruct((NW * PPW,), jnp.float32),
        ),
        scratch_types=[
            pltpu.VMEM((32, 128), jnp.float32),
            pltpu.VMEM((32, 128), jnp.float32),
            pltpu.VMEM((4096,), jnp.float32),
            pltpu.VMEM((4096,), jnp.float32),
            pltpu.VMEM((PPW,), jnp.int32),
            pltpu.VMEM((PPW,), jnp.float32),
            pltpu.VMEM((16384,), jnp.float32),
            pltpu.VMEM_SHARED((NUM_FEATURES,), jnp.float32),
            pltpu.SemaphoreType.DMA,
            pltpu.SemaphoreType.DMA,
            pltpu.SemaphoreType.DMA,
        ],
        compiler_params=pltpu.CompilerParams(
            use_tc_tiling_on_sc=True, needs_layout_passes=False),
    )(_p1_body)
    lin, wg_all = p1(emb.T, tailf, linear_w, idp)

    p2 = functools.partial(
        pl.kernel,
        mesh=mesh,
        out_type=jax.ShapeDtypeStruct((BATCH,), jnp.float32),
        scratch_types=[
            pltpu.VMEM((IPW,), jnp.int32),
            pltpu.VMEM((IPW, EMBED_DIM), jnp.float32),
            pltpu.VMEM((PPW,), jnp.float32),
            pltpu.VMEM((PPW,), jnp.float32),
            pltpu.VMEM((16,), jnp.float32),
            pltpu.VMEM((SPW,), jnp.float32),
            pltpu.SemaphoreType.DMA,
        ],
        compiler_params=pltpu.CompilerParams(
            use_tc_tiling_on_sc=False, needs_layout_passes=False),
    )(_p2_body)
    return p2(lin.reshape(NUM_FEATURES, EMBED_DIM),
              wg_all.reshape(NW, PPW), idf, vp, bias16)


def kernel(feature_ids_batch, feature_values_batch, bias, linear_w, emb):
    ids = feature_ids_batch.astype(jnp.int32)
    vals = feature_values_batch.astype(jnp.float32)
    pad_i = jnp.zeros((BATCH, 32 - FIELDS), jnp.int32)
    pad_v = jnp.zeros((BATCH, 32 - FIELDS), jnp.float32)
    idf = ids.reshape(NW, IPW)
    idp = jnp.concatenate([ids, pad_i], axis=1).reshape(NW, PPW)
    vp = jnp.concatenate([vals, pad_v], axis=1).reshape(NW, PPW)
    bias16 = jnp.broadcast_to(bias, (16,))
    tailf = emb[TAIL_R:, :].reshape(-1)
    out = _fm(emb, linear_w, idf, idp, vp, bias16, tailf)
    return out.reshape(BATCH, 1)


# speedup vs baseline: 2.4925x; 1.0018x over previous
import functools

import jax
import jax.numpy as jnp
from jax import lax
from jax.experimental import pallas as pl
from jax.experimental.pallas import tpu as pltpu
from jax.experimental.pallas import tpu_sc as plsc

NUM_FEATURES = 1000000
EMBED_DIM = 32
BATCH = 4096
FIELDS = 26
NW = 32
SPW = BATCH // NW
IPW = SPW * FIELDS
PPW = SPW * 32
NBLK = NUM_FEATURES // 128
NPAIR = NBLK // 2
PBW = NPAIR // NW
PREM = NPAIR - PBW * NW
TAIL_R = NBLK * 128
TAIL_W = (NUM_FEATURES - TAIL_R) * EMBED_DIM
WSTRIPE = 62496
WLAST = NUM_FEATURES - 15 * WSTRIPE


def _p1_body(embT_h, tailf_h, w_h, idp_h, lin_h, wg_h,
             in0, in1, ob0, ob1, idp_v, wbuf_v, wbn_v, wsp, sin, sout, semw):
    wid = lax.axis_index("s") * 2 + lax.axis_index("c")
    sid = lax.axis_index("s")
    start = wid * PBW + jnp.minimum(wid, PREM)
    cnt = jnp.where(wid < PREM, PBW + 1, PBW)
    lane = lax.iota(jnp.int32, 16)

    @pl.when(sid < 15)
    def _():
        for j in range(4):
            o = pl.multiple_of(sid * 65536 + j * 16384, 16384)
            pltpu.sync_copy(w_h.at[pl.ds(o, 16384)], wbn_v)
            pltpu.sync_copy(wbn_v, wsp.at[pl.ds(o, 16384)])

    @pl.when(sid == 15)
    def _():
        pltpu.sync_copy(w_h.at[pl.ds(983040, 16384)], wbn_v)
        pltpu.sync_copy(wbn_v, wsp.at[pl.ds(983040, 16384)])
        pltpu.sync_copy(w_h.at[pl.ds(999424, 576)], wbn_v.at[pl.ds(0, 576)])
        pltpu.sync_copy(wbn_v.at[pl.ds(0, 576)], wsp.at[pl.ds(999424, 576)])

    pltpu.sync_copy(idp_h.at[wid], idp_v)
    plsc.subcore_barrier()

    def fire_w(k, _):
        o = pl.multiple_of(k * 16, 16)
        iv = idp_v[pl.ds(o, 16)]
        pltpu.async_copy(wsp.at[iv], wbuf_v.at[pl.ds(o, 16)], semw)
        return 0

    lax.fori_loop(0, PPW // 16, fire_w, 0)

    def fire_in(b, buf):
        pltpu.async_copy(
            embT_h.at[:, pl.ds(pl.multiple_of(b * 256, 256), 256)], buf, sin)

    def wait_in(buf):
        pltpu.make_async_copy(embT_h.at[:, pl.ds(0, 256)], buf, sin).wait()

    def fire_out(b, obuf):
        pltpu.async_copy(
            obuf, lin_h.at[pl.ds(pl.multiple_of(b * 8192, 8192), 8192)], sout)

    def wait_out(obuf):
        pltpu.make_async_copy(lin_h.at[pl.ds(0, 8192)], obuf, sout).wait()

    fire_in(start, in0)

    def pair(g2, _):
        for par in (0, 1):
            bufi = in0 if par == 0 else in1
            bufo = ob0 if par == 0 else ob1
            b = start + g2 * 2 + par

            @pl.when(b < start + cnt)
            def _():
                wait_in(bufi)

                @pl.when(b + 1 < start + cnt)
                def _():
                    fire_in(b + 1, in1 if par == 0 else in0)

                def c0loop(c04, _):
                    for u in range(4):
                        cv = (c04 * 4 + u + lane) & 31
                        for blk in range(2):
                            for rg in range(8):
                                rv = rg * 16 + lane
                                e = plsc.load_gather(
                                    bufi, [cv, blk * 128 + rv])
                                plsc.store_scatter(
                                    bufo, [blk * 4096 + rv * 32 + cv], e)
                    return 0

                lax.fori_loop(0, 8, c0loop, 0)

                @pl.when(b - 2 >= start)
                def _():
                    wait_out(bufo)

                fire_out(b, bufo)
        return 0

    lax.fori_loop(0, (PBW + 2) // 2, pair, 0)
    wait_out(ob0)
    wait_out(ob1)

    @pl.when(wid == 0)
    def _():
        pltpu.sync_copy(tailf_h, ob0.at[pl.ds(0, TAIL_W)])
        pltpu.sync_copy(ob0.at[pl.ds(0, TAIL_W)],
                        lin_h.at[pl.ds(TAIL_R * EMBED_DIM, TAIL_W)])

    pltpu.make_async_copy(wsp.at[pl.ds(0, PPW)], wbuf_v, semw).wait()
    pltpu.sync_copy(wbuf_v, wg_h.at[pl.ds(pl.multiple_of(wid * PPW, PPW), PPW)])


def _p2_body(emb_h, wg2_h, idf_h, vp_h, bias_h, out_h,
             idx_v, rows_v, wg_v, vals_v, bias_v, out_v, sem_r):
    wid = lax.axis_index("s") * 2 + lax.axis_index("c")
    base = pl.multiple_of(wid * SPW, SPW)

    pltpu.sync_copy(idf_h.at[wid], idx_v)
    pltpu.sync_copy(wg2_h.at[wid], wg_v)
    pltpu.sync_copy(vp_h.at[wid], vals_v)
    pltpu.sync_copy(bias_h, bias_v)

    def fire_rows(k, _):
        o = pl.multiple_of(k * 16, 16)
        iv = idx_v[pl.ds(o, 16)]
        pltpu.async_copy(emb_h.at[iv], rows_v.at[pl.ds(o, 16)], sem_r)
        return 0

    lax.fori_loop(0, IPW // 16, fire_rows, 0)
    pltpu.make_async_copy(emb_h.at[pl.ds(0, IPW)], rows_v, sem_r).wait()

    lane = lax.iota(jnp.int32, 16)
    perms = [lane ^ 1, lane ^ 2, lane ^ 4, lane ^ 8]

    def body(s, xvec):
        off = pl.multiple_of(s * 32, 32)
        v0 = vals_v[pl.ds(off, 16)]
        v1 = vals_v[pl.ds(off + 16, 16)]
        w0 = wg_v[pl.ds(off, 16)]
        w1 = wg_v[pl.ds(off + 16, 16)]
        i0 = s * FIELDS
        acc0 = jnp.zeros((16,), jnp.float32)
        acc1 = jnp.zeros((16,), jnp.float32)
        ssqv = jnp.zeros((16,), jnp.float32)
        for f in range(FIELDS):
            e0 = rows_v[i0 + f, pl.ds(0, 16)]
            e1 = rows_v[i0 + f, pl.ds(16, 16)]
            src = v0 if f < 16 else v1
            vb = src.at[jnp.full((16,), f % 16, jnp.int32)].get(
                mode="promise_in_bounds")
            s0 = e0 * vb
            s1 = e1 * vb
            acc0 = acc0 + s0
            acc1 = acc1 + s1
            ssqv = ssqv + (s0 * s0 + s1 * s1)
        sqv = acc0 * acc0 + acc1 * acc1
        linv = w0 * v0 + w1 * v1
        xv = linv + 0.5 * (sqv - ssqv)
        for p in perms:
            xv = xv + xv.at[p].get(mode="promise_in_bounds")
        xvec = jnp.where(lane == (s % 16), xv, xvec)

        @pl.when(s % 16 == 15)
        def _():
            out_v[pl.ds(pl.multiple_of((s // 16) * 16, 16), 16)] = xvec

        return xvec

    lax.fori_loop(0, SPW, body, jnp.zeros((16,), jnp.float32))

    bvec = bias_v[...]
    for k in range(SPW // 16):
        x = out_v[pl.ds(k * 16, 16)]
        y = 1.0 / (1.0 + jnp.exp(-(x + bvec)))
        out_v[pl.ds(k * 16, 16)] = y
    pltpu.sync_copy(out_v, out_h.at[pl.ds(base, SPW)])


@jax.jit
def _fm(emb, linear_w, idf, idp, vp, bias16, tailf):
    mesh = plsc.VectorSubcoreMesh(core_axis_name="c", subcore_axis_name="s")
    p1 = functools.partial(
        pl.kernel,
        mesh=mesh,
        out_type=(
            jax.ShapeDtypeStruct((NUM_FEATURES * EMBED_DIM,), jnp.float32),
            jax.ShapeDtypeStruct((NW * PPW,), jnp.float32),
        ),
        scratch_types=[
            pltpu.VMEM((32, 256), jnp.float32),
            pltpu.VMEM((32, 256), jnp.float32),
            pltpu.VMEM((8192,), jnp.float32),
            pltpu.VMEM((8192,), jnp.float32),
            pltpu.VMEM((PPW,), jnp.int32),
            pltpu.VMEM((PPW,), jnp.float32),
            pltpu.VMEM((16384,), jnp.float32),
            pltpu.VMEM_SHARED((NUM_FEATURES,), jnp.float32),
            pltpu.SemaphoreType.DMA,
            pltpu.SemaphoreType.DMA,
            pltpu.SemaphoreType.DMA,
        ],
        compiler_params=pltpu.CompilerParams(
            use_tc_tiling_on_sc=True, needs_layout_passes=False),
    )(_p1_body)
    lin, wg_all = p1(emb.T, tailf, linear_w, idp)

    p2 = functools.partial(
        pl.kernel,
        mesh=mesh,
        out_type=jax.ShapeDtypeStruct((BATCH,), jnp.float32),
        scratch_types=[
            pltpu.VMEM((IPW,), jnp.int32),
            pltpu.VMEM((IPW, EMBED_DIM), jnp.float32),
            pltpu.VMEM((PPW,), jnp.float32),
            pltpu.VMEM((PPW,), jnp.float32),
            pltpu.VMEM((16,), jnp.float32),
            pltpu.VMEM((SPW,), jnp.float32),
            pltpu.SemaphoreType.DMA,
        ],
        compiler_params=pltpu.CompilerParams(
            use_tc_tiling_on_sc=False, needs_layout_passes=False),
    )(_p2_body)
    return p2(lin.reshape(NUM_FEATURES, EMBED_DIM),
              wg_all.reshape(NW, PPW), idf, vp, bias16)


def kernel(feature_ids_batch, feature_values_batch, bias, linear_w, emb):
    ids = feature_ids_batch.astype(jnp.int32)
    vals = feature_values_batch.astype(jnp.float32)
    pad_i = jnp.zeros((BATCH, 32 - FIELDS), jnp.int32)
    pad_v = jnp.zeros((BATCH, 32 - FIELDS), jnp.float32)
    idf = ids.reshape(NW, IPW)
    idp = jnp.concatenate([ids, pad_i], axis=1).reshape(NW, PPW)
    vp = jnp.concatenate([vals, pad_v], axis=1).reshape(NW, PPW)
    bias16 = jnp.broadcast_to(bias, (16,))
    tailf = emb[TAIL_R:, :].reshape(-1)
    out = _fm(emb, linear_w, idf, idp, vp, bias16, tailf)
    return out.reshape(BATCH, 1)
